# baseline (speedup 1.0000x reference)
import functools

import jax
import jax.numpy as jnp
from jax import lax
from jax.experimental import pallas as pl
from jax.experimental.pallas import tpu as pltpu
from jax.experimental.pallas import tpu_sc as plsc

TOP_K = 8
L = 16
NC, NS = 2, 16
NW = NC * NS


def _matmul_body_first(x_ref, w_ref, b_ref, out_ref, outT_ref):
    acc = jnp.dot(x_ref[...], w_ref[...], preferred_element_type=jnp.float32)
    acc = acc + b_ref[...]
    out_ref[...] = acc
    outT_ref[...] = acc.T


def _matmul_body_alias(x_ref, w_ref, b_ref, prev_ref, out_ref, outT_ref):
    del prev_ref
    _matmul_body_first(x_ref, w_ref, b_ref, out_ref, outT_ref)


def _router_logits(x, w_t, bias2d, prev, block_tokens, chunk_tokens, chunk):
    tokens, hidden = x.shape
    experts = w_t.shape[1]
    blocks_per_chunk = chunk_tokens // block_tokens
    base = chunk * blocks_per_chunk
    ins = [x, w_t, bias2d]
    in_specs = [
        pl.BlockSpec((block_tokens, hidden), lambda i: (base + i, 0)),
        pl.BlockSpec((hidden, experts), lambda i: (0, 0)),
        pl.BlockSpec((1, experts), lambda i: (0, 0)),
    ]
    aliases = {}
    body = _matmul_body_first
    if prev is not None:
        ins.append(prev)
        in_specs.append(pl.BlockSpec(memory_space=pl.ANY))
        aliases = {3: 0}
        body = _matmul_body_alias
        logits_shape = (tokens, experts)
        logits_spec = pl.BlockSpec((block_tokens, experts),
                                   lambda i: (base + i, 0))
    else:
        logits_shape = (chunk_tokens, experts)
        logits_spec = pl.BlockSpec((block_tokens, experts), lambda i: (i, 0))
    return pl.pallas_call(
        body,
        grid=(blocks_per_chunk,),
        in_specs=in_specs,
        out_specs=[
            logits_spec,
            pl.BlockSpec((experts, block_tokens), lambda i: (0, i)),
        ],
        out_shape=[
            jax.ShapeDtypeStruct(logits_shape, jnp.float32),
            jax.ShapeDtypeStruct((experts, chunk_tokens), jnp.float32),
        ],
        input_output_aliases=aliases,
        compiler_params=pltpu.CompilerParams(
            dimension_semantics=("arbitrary",),
        ),
    )(*ins)


def _make_topk_sc(tokens, experts):
    tw = tokens // NW
    groups = tw // L

    def body(lt_ref, scores_ref, idx_ref, tile, obuf_s, obuf_i, sem):
        neg_inf = jnp.full((L,), -jnp.inf, dtype=jnp.float32)
        c = lax.axis_index("c")
        s = lax.axis_index("s")
        wid = s * NC + c
        t0 = wid * tw
        copies = [
            pltpu.async_copy(lt_ref.at[j, pl.ds(t0, tw)],
                             tile.at[pl.ds(j * tw, tw)], sem)
            for j in range(experts)
        ]
        for cp in copies:
            cp.wait()
        lanes = lax.iota(jnp.int32, L)

        def group_body(g, carry):
            col = g * L
            toks = col + lanes
            toks_k = toks * TOP_K
            ms, ixs = [], []
            for _ in range(TOP_K):
                accs = []
                for blk in range(8):
                    j0 = blk * 8
                    m = tile[pl.ds(j0 * tw + col, L)]
                    ix = jnp.full((L,), j0, dtype=jnp.int32)
                    for j in range(j0 + 1, j0 + 8):
                        v = tile[pl.ds(j * tw + col, L)]
                        p = v > m
                        m = jnp.where(p, v, m)
                        ix = jnp.where(p, jnp.full((L,), j, dtype=jnp.int32), ix)
                    accs.append((m, ix))
                while len(accs) > 1:
                    nxt = []
                    for a, b in zip(accs[0::2], accs[1::2]):
                        p = b[0] > a[0]
                        nxt.append((jnp.where(p, b[0], a[0]),
                                    jnp.where(p, b[1], a[1])))
                    accs = nxt
                m, ix = accs[0]
                ms.append(m)
                ixs.append(ix)
                plsc.store_scatter(tile, [ix * tw + toks], neg_inf)
            es = [jnp.exp(mm - ms[0]) for mm in ms]
            tot = es[0]
            for e in es[1:]:
                tot = tot + e
            inv = 1.0 / tot
            for r in range(TOP_K):
                plsc.store_scatter(obuf_s, [toks_k + r], es[r] * inv)
                plsc.store_scatter(obuf_i, [toks_k + r], ixs[r])
            return carry

        lax.fori_loop(0, groups, group_body, 0)
        pltpu.sync_copy(obuf_s, scores_ref.at[pl.ds(t0 * TOP_K, tw * TOP_K)])
        pltpu.sync_copy(obuf_i, idx_ref.at[pl.ds(t0 * TOP_K, tw * TOP_K)])

    return pl.kernel(
        body,
        out_type=[
            jax.ShapeDtypeStruct((tokens * TOP_K,), jnp.float32),
            jax.ShapeDtypeStruct((tokens * TOP_K,), jnp.int32),
        ],
        mesh=plsc.VectorSubcoreMesh(core_axis_name="c", subcore_axis_name="s"),
        compiler_params=pltpu.CompilerParams(needs_layout_passes=False),
        scratch_types=[
            pltpu.VMEM((experts * tw,), jnp.float32),
            pltpu.VMEM((tw * TOP_K,), jnp.float32),
            pltpu.VMEM((tw * TOP_K,), jnp.int32),
            pltpu.SemaphoreType.DMA,
        ],
    )


NUM_CHUNKS = 4


@jax.jit
def kernel(hidden_states, weight, bias):
    tokens, _ = hidden_states.shape
    experts = weight.shape[0]
    w_t = weight.T
    bias2d = bias.reshape(1, experts)
    tc = tokens // NUM_CHUNKS
    topk_sc = _make_topk_sc(tc, experts)
    logits = jnp.zeros((tokens, experts), jnp.float32)
    scores_flat = jnp.zeros((tokens * TOP_K,), jnp.float32)
    idx_flat = jnp.zeros((tokens * TOP_K,), jnp.int32)
    for i in range(NUM_CHUNKS):
        logits_i, logits_t_i = _router_logits(hidden_states, w_t, bias2d,
                                              None, block_tokens=512,
                                              chunk_tokens=tc, chunk=i)
        scores_i, idx_i = topk_sc(logits_t_i)
        logits = jax.lax.dynamic_update_slice(logits, logits_i, (i * tc, 0))
        scores_flat = jax.lax.dynamic_update_slice(scores_flat, scores_i,
                                                   (i * tc * TOP_K,))
        idx_flat = jax.lax.dynamic_update_slice(idx_flat, idx_i,
                                                (i * tc * TOP_K,))
    return (logits,
            scores_flat.reshape(tokens, TOP_K),
            idx_flat.reshape(tokens, TOP_K))


# speedup vs baseline: 1.0893x; 1.0893x over previous
import functools

import jax
import jax.numpy as jnp
from jax import lax
from jax.experimental import pallas as pl
from jax.experimental.pallas import tpu as pltpu
from jax.experimental.pallas import tpu_sc as plsc

TOP_K = 8
L = 16
NC, NS = 2, 16
NW = NC * NS


def _matmul_body_first(x_ref, w_ref, b_ref, out_ref, outT_ref):
    acc = jnp.dot(x_ref[...], w_ref[...], preferred_element_type=jnp.float32)
    acc = acc + b_ref[...]
    out_ref[...] = acc
    outT_ref[...] = acc.T


def _matmul_body_alias(x_ref, w_ref, b_ref, prev_ref, out_ref, outT_ref):
    del prev_ref
    _matmul_body_first(x_ref, w_ref, b_ref, out_ref, outT_ref)


def _router_logits(x, w_t, bias2d, prev, block_tokens, chunk_tokens, chunk):
    tokens, hidden = x.shape
    experts = w_t.shape[1]
    blocks_per_chunk = chunk_tokens // block_tokens
    base = chunk * blocks_per_chunk
    ins = [x, w_t, bias2d]
    in_specs = [
        pl.BlockSpec((block_tokens, hidden), lambda i: (base + i, 0)),
        pl.BlockSpec((hidden, experts), lambda i: (0, 0)),
        pl.BlockSpec((1, experts), lambda i: (0, 0)),
    ]
    aliases = {}
    body = _matmul_body_first
    if prev is not None:
        ins.append(prev)
        in_specs.append(pl.BlockSpec(memory_space=pl.ANY))
        aliases = {3: 0}
        body = _matmul_body_alias
        logits_shape = (tokens, experts)
        logits_spec = pl.BlockSpec((block_tokens, experts),
                                   lambda i: (base + i, 0))
    else:
        logits_shape = (chunk_tokens, experts)
        logits_spec = pl.BlockSpec((block_tokens, experts), lambda i: (i, 0))
    return pl.pallas_call(
        body,
        grid=(blocks_per_chunk,),
        in_specs=in_specs,
        out_specs=[
            logits_spec,
            pl.BlockSpec((experts, block_tokens), lambda i: (0, i)),
        ],
        out_shape=[
            jax.ShapeDtypeStruct(logits_shape, jnp.float32),
            jax.ShapeDtypeStruct((experts, chunk_tokens), jnp.float32),
        ],
        input_output_aliases=aliases,
        compiler_params=pltpu.CompilerParams(
            dimension_semantics=("arbitrary",),
        ),
    )(*ins)


def _make_topk_sc(tokens, experts):
    tw = tokens // NW
    groups = tw // L

    def body(lt_ref, scores_ref, idx_ref, tile, obuf_s, obuf_i, sem):
        neg_inf = jnp.full((L,), -jnp.inf, dtype=jnp.float32)
        c = lax.axis_index("c")
        s = lax.axis_index("s")
        wid = s * NC + c
        t0 = wid * tw
        copies = [
            pltpu.async_copy(lt_ref.at[j, pl.ds(t0, tw)],
                             tile.at[pl.ds(j * tw, tw)], sem)
            for j in range(experts)
        ]
        for cp in copies:
            cp.wait()
        lanes = lax.iota(jnp.int32, L)

        def group_body(g, carry):
            col = g * L
            toks = col + lanes
            toks_k = toks * TOP_K
            ms, ixs = [], []
            for _ in range(TOP_K):
                accs = []
                for blk in range(8):
                    j0 = blk * 8
                    m = tile[pl.ds(j0 * tw + col, L)]
                    ix = jnp.full((L,), j0, dtype=jnp.int32)
                    for j in range(j0 + 1, j0 + 8):
                        v = tile[pl.ds(j * tw + col, L)]
                        p = v > m
                        m = jnp.where(p, v, m)
                        ix = jnp.where(p, jnp.full((L,), j, dtype=jnp.int32), ix)
                    accs.append((m, ix))
                while len(accs) > 1:
                    nxt = []
                    for a, b in zip(accs[0::2], accs[1::2]):
                        p = b[0] > a[0]
                        nxt.append((jnp.where(p, b[0], a[0]),
                                    jnp.where(p, b[1], a[1])))
                    accs = nxt
                m, ix = accs[0]
                ms.append(m)
                ixs.append(ix)
                plsc.store_scatter(tile, [ix * tw + toks], neg_inf)
            es = [jnp.exp(mm - ms[0]) for mm in ms]
            tot = es[0]
            for e in es[1:]:
                tot = tot + e
            inv = 1.0 / tot
            for r in range(TOP_K):
                plsc.store_scatter(obuf_s, [toks_k + r], es[r] * inv)
                plsc.store_scatter(obuf_i, [toks_k + r], ixs[r])
            return carry

        lax.fori_loop(0, groups, group_body, 0)
        pltpu.sync_copy(obuf_s, scores_ref.at[pl.ds(t0 * TOP_K, tw * TOP_K)])
        pltpu.sync_copy(obuf_i, idx_ref.at[pl.ds(t0 * TOP_K, tw * TOP_K)])

    return pl.kernel(
        body,
        out_type=[
            jax.ShapeDtypeStruct((tokens * TOP_K,), jnp.float32),
            jax.ShapeDtypeStruct((tokens * TOP_K,), jnp.int32),
        ],
        mesh=plsc.VectorSubcoreMesh(core_axis_name="c", subcore_axis_name="s"),
        compiler_params=pltpu.CompilerParams(needs_layout_passes=False),
        scratch_types=[
            pltpu.VMEM((experts * tw,), jnp.float32),
            pltpu.VMEM((tw * TOP_K,), jnp.float32),
            pltpu.VMEM((tw * TOP_K,), jnp.int32),
            pltpu.SemaphoreType.DMA,
        ],
    )


NUM_CHUNKS = 1


@jax.jit
def kernel(hidden_states, weight, bias):
    tokens, _ = hidden_states.shape
    experts = weight.shape[0]
    w_t = weight.T
    bias2d = bias.reshape(1, experts)
    tc = tokens // NUM_CHUNKS
    topk_sc = _make_topk_sc(tc, experts)
    logits_chunks, scores_chunks, idx_chunks = [], [], []
    for i in range(NUM_CHUNKS):
        logits_i, logits_t_i = _router_logits(hidden_states, w_t, bias2d,
                                              None, block_tokens=512,
                                              chunk_tokens=tc, chunk=i)
        scores_i, idx_i = topk_sc(logits_t_i)
        logits_chunks.append(logits_i)
        scores_chunks.append(scores_i)
        idx_chunks.append(idx_i)
    logits = (logits_chunks[0] if NUM_CHUNKS == 1
              else jnp.concatenate(logits_chunks, axis=0))
    scores = (scores_chunks[0] if NUM_CHUNKS == 1
              else jnp.concatenate(scores_chunks))
    indices = (idx_chunks[0] if NUM_CHUNKS == 1
               else jnp.concatenate(idx_chunks))
    return (logits,
            scores.reshape(tokens, TOP_K),
            indices.reshape(tokens, TOP_K))


# speedup vs baseline: 1.1054x; 1.0147x over previous
import functools

import jax
import jax.numpy as jnp
from jax import lax
from jax.experimental import pallas as pl
from jax.experimental.pallas import tpu as pltpu
from jax.experimental.pallas import tpu_sc as plsc

TOP_K = 8
L = 16
NC, NS = 2, 16
NW = NC * NS


def _matmul_body_first(x_ref, w_ref, b_ref, out_ref, outT_ref):
    acc = jnp.dot(x_ref[...], w_ref[...], preferred_element_type=jnp.float32)
    acc = acc + b_ref[...]
    out_ref[...] = acc
    outT_ref[...] = acc.T


def _matmul_body_alias(x_ref, w_ref, b_ref, prev_ref, out_ref, outT_ref):
    del prev_ref
    _matmul_body_first(x_ref, w_ref, b_ref, out_ref, outT_ref)


def _router_logits(x, w_t, bias2d, prev, block_tokens, chunk_tokens, chunk):
    tokens, hidden = x.shape
    experts = w_t.shape[1]
    blocks_per_chunk = chunk_tokens // block_tokens
    base = chunk * blocks_per_chunk
    ins = [x, w_t, bias2d]
    in_specs = [
        pl.BlockSpec((block_tokens, hidden), lambda i: (base + i, 0)),
        pl.BlockSpec((hidden, experts), lambda i: (0, 0)),
        pl.BlockSpec((1, experts), lambda i: (0, 0)),
    ]
    aliases = {}
    body = _matmul_body_first
    if prev is not None:
        ins.append(prev)
        in_specs.append(pl.BlockSpec(memory_space=pl.ANY))
        aliases = {3: 0}
        body = _matmul_body_alias
        logits_shape = (tokens, experts)
        logits_spec = pl.BlockSpec((block_tokens, experts),
                                   lambda i: (base + i, 0))
    else:
        logits_shape = (chunk_tokens, experts)
        logits_spec = pl.BlockSpec((block_tokens, experts), lambda i: (i, 0))
    return pl.pallas_call(
        body,
        grid=(blocks_per_chunk,),
        in_specs=in_specs,
        out_specs=[
            logits_spec,
            pl.BlockSpec((experts, block_tokens), lambda i: (0, i)),
        ],
        out_shape=[
            jax.ShapeDtypeStruct(logits_shape, jnp.float32),
            jax.ShapeDtypeStruct((experts, chunk_tokens), jnp.float32),
        ],
        input_output_aliases=aliases,
        compiler_params=pltpu.CompilerParams(
            dimension_semantics=("arbitrary",),
        ),
    )(*ins)


def _make_topk_sc(tokens, experts):
    tw = tokens // NW
    sub = 128
    groups = sub // L

    def body(lt_ref, scores_ref, idx_ref, tile, obuf_s, obuf_i, sem):
        neg_inf = jnp.full((L,), -jnp.inf, dtype=jnp.float32)
        c = lax.axis_index("c")
        s = lax.axis_index("s")
        wid = s * NC + c
        t0 = wid * tw
        copies = [
            pltpu.async_copy(lt_ref.at[j, pl.ds(t0, tw)],
                             tile.at[pl.ds(j * tw, tw)], sem)
            for j in range(experts)
        ]
        for cp in copies:
            cp.wait()
        lanes = lax.iota(jnp.int32, L)

        def make_group_body(sb):
            def group_body(g, carry):
                col = sb * sub + g * L
                toks = col + lanes
                otoks = g * L + lanes
                ms, ixs = [], []
                for _ in range(TOP_K):
                    accs = []
                    for blk in range(8):
                        j0 = blk * 8
                        m = tile[pl.ds(j0 * tw + col, L)]
                        ix = jnp.full((L,), j0, dtype=jnp.int32)
                        for j in range(j0 + 1, j0 + 8):
                            v = tile[pl.ds(j * tw + col, L)]
                            p = v > m
                            m = jnp.where(p, v, m)
                            ix = jnp.where(p, jnp.full((L,), j, dtype=jnp.int32),
                                           ix)
                        accs.append((m, ix))
                    while len(accs) > 1:
                        nxt = []
                        for a, b in zip(accs[0::2], accs[1::2]):
                            p = b[0] > a[0]
                            nxt.append((jnp.where(p, b[0], a[0]),
                                        jnp.where(p, b[1], a[1])))
                        accs = nxt
                    m, ix = accs[0]
                    ms.append(m)
                    ixs.append(ix)
                    plsc.store_scatter(tile, [ix * tw + toks], neg_inf)
                es = [jnp.exp(mm - ms[0]) for mm in ms]
                tot = es[0]
                for e in es[1:]:
                    tot = tot + e
                inv = 1.0 / tot
                for r in range(TOP_K):
                    rcol = jnp.full((L,), r, dtype=jnp.int32)
                    plsc.store_scatter(obuf_s, [otoks, rcol], es[r] * inv)
                    plsc.store_scatter(obuf_i, [otoks, rcol], ixs[r])
                return carry
            return group_body

        for sb in range(tw // sub):
            lax.fori_loop(0, groups, make_group_body(sb), 0)
            pltpu.sync_copy(obuf_s, scores_ref.at[pl.ds(t0 + sb * sub, sub), :])
            pltpu.sync_copy(obuf_i, idx_ref.at[pl.ds(t0 + sb * sub, sub), :])

    return pl.kernel(
        body,
        out_type=[
            jax.ShapeDtypeStruct((tokens, TOP_K), jnp.float32),
            jax.ShapeDtypeStruct((tokens, TOP_K), jnp.int32),
        ],
        mesh=plsc.VectorSubcoreMesh(core_axis_name="c", subcore_axis_name="s"),
        compiler_params=pltpu.CompilerParams(needs_layout_passes=False),
        scratch_types=[
            pltpu.VMEM((experts * tw,), jnp.float32),
            pltpu.VMEM((sub, TOP_K), jnp.float32),
            pltpu.VMEM((sub, TOP_K), jnp.int32),
            pltpu.SemaphoreType.DMA,
        ],
    )


def _pack_body(s_ref, i_ref, so_ref, io_ref):
    bt = so_ref.shape[0]
    so_ref[...] = s_ref[...].reshape(bt, TOP_K)
    io_ref[...] = i_ref[...].reshape(bt, TOP_K)


def _pack_outputs(scores_flat, idx_flat, tokens, block_tokens=2048):
    return pl.pallas_call(
        _pack_body,
        grid=(tokens // block_tokens,),
        in_specs=[pl.BlockSpec((block_tokens * TOP_K,), lambda i: (i,)),
                  pl.BlockSpec((block_tokens * TOP_K,), lambda i: (i,))],
        out_specs=[pl.BlockSpec((block_tokens, TOP_K), lambda i: (i, 0)),
                   pl.BlockSpec((block_tokens, TOP_K), lambda i: (i, 0))],
        out_shape=[jax.ShapeDtypeStruct((tokens, TOP_K), jnp.float32),
                   jax.ShapeDtypeStruct((tokens, TOP_K), jnp.int32)],
        compiler_params=pltpu.CompilerParams(
            dimension_semantics=("arbitrary",),
        ),
    )(scores_flat, idx_flat)


NUM_CHUNKS = 1


@jax.jit
def kernel(hidden_states, weight, bias):
    tokens, _ = hidden_states.shape
    experts = weight.shape[0]
    w_t = weight.T
    bias2d = bias.reshape(1, experts)
    logits, logits_t = _router_logits(hidden_states, w_t, bias2d,
                                      None, block_tokens=512,
                                      chunk_tokens=tokens, chunk=0)
    scores, indices = _make_topk_sc(tokens, experts)(logits_t)
    return logits, scores, indices


# speedup vs baseline: 1.1321x; 1.0242x over previous
import functools

import jax
import jax.numpy as jnp
from jax import lax
from jax.experimental import pallas as pl
from jax.experimental.pallas import tpu as pltpu
from jax.experimental.pallas import tpu_sc as plsc

TOP_K = 8
L = 16
NC, NS = 2, 16
NW = NC * NS


def _matmul_body_first(x_ref, w_ref, b_ref, out_ref, outT_ref):
    acc = jnp.dot(x_ref[...], w_ref[...], preferred_element_type=jnp.float32)
    acc = acc + b_ref[...]
    out_ref[...] = acc
    outT_ref[...] = acc.T


def _matmul_body_alias(x_ref, w_ref, b_ref, prev_ref, out_ref, outT_ref):
    del prev_ref
    _matmul_body_first(x_ref, w_ref, b_ref, out_ref, outT_ref)


def _router_logits(x, w_t, bias2d, prev, block_tokens, chunk_tokens, chunk):
    tokens, hidden = x.shape
    experts = w_t.shape[1]
    blocks_per_chunk = chunk_tokens // block_tokens
    base = chunk * blocks_per_chunk
    ins = [x, w_t, bias2d]
    in_specs = [
        pl.BlockSpec((block_tokens, hidden), lambda i: (base + i, 0)),
        pl.BlockSpec((hidden, experts), lambda i: (0, 0)),
        pl.BlockSpec((1, experts), lambda i: (0, 0)),
    ]
    aliases = {}
    body = _matmul_body_first
    if prev is not None:
        ins.append(prev)
        in_specs.append(pl.BlockSpec(memory_space=pl.ANY))
        aliases = {3: 0}
        body = _matmul_body_alias
        logits_shape = (tokens, experts)
        logits_spec = pl.BlockSpec((block_tokens, experts),
                                   lambda i: (base + i, 0))
    else:
        logits_shape = (chunk_tokens, experts)
        logits_spec = pl.BlockSpec((block_tokens, experts), lambda i: (i, 0))
    return pl.pallas_call(
        body,
        grid=(blocks_per_chunk,),
        in_specs=in_specs,
        out_specs=[
            logits_spec,
            pl.BlockSpec((experts, block_tokens), lambda i: (0, i)),
        ],
        out_shape=[
            jax.ShapeDtypeStruct(logits_shape, jnp.float32),
            jax.ShapeDtypeStruct((experts, chunk_tokens), jnp.float32),
        ],
        input_output_aliases=aliases,
        compiler_params=pltpu.CompilerParams(
            dimension_semantics=("arbitrary",),
        ),
    )(*ins)


def _make_topk_sc(tokens, experts):
    tw = tokens // NW
    sub = 128
    groups = sub // L

    def body(lt_ref, scores_ref, idx_ref, tile, obuf_s, obuf_i, sem, sems):
        neg_inf = jnp.full((L,), -jnp.inf, dtype=jnp.float32)
        c = lax.axis_index("c")
        s = lax.axis_index("s")
        wid = s * NC + c
        t0 = wid * tw
        copies = [
            pltpu.async_copy(lt_ref.at[j, pl.ds(t0, tw)],
                             tile.at[pl.ds(j * tw, tw)], sem)
            for j in range(experts)
        ]
        for cp in copies:
            cp.wait()
        lanes = lax.iota(jnp.int32, L)

        def make_group_body(sb, half):
            def group_body(g, carry):
                col = sb * sub + g * L
                toks = col + lanes
                otoks = half * sub + g * L + lanes
                ms, ixs = [], []
                for _ in range(TOP_K):
                    accs = []
                    for blk in range(8):
                        j0 = blk * 8
                        m = tile[pl.ds(j0 * tw + col, L)]
                        ix = jnp.full((L,), j0, dtype=jnp.int32)
                        for j in range(j0 + 1, j0 + 8):
                            v = tile[pl.ds(j * tw + col, L)]
                            p = v > m
                            m = jnp.where(p, v, m)
                            ix = jnp.where(p, jnp.full((L,), j, dtype=jnp.int32),
                                           ix)
                        accs.append((m, ix))
                    while len(accs) > 1:
                        nxt = []
                        for a, b in zip(accs[0::2], accs[1::2]):
                            p = b[0] > a[0]
                            nxt.append((jnp.where(p, b[0], a[0]),
                                        jnp.where(p, b[1], a[1])))
                        accs = nxt
                    m, ix = accs[0]
                    ms.append(m)
                    ixs.append(ix)
                    plsc.store_scatter(tile, [ix * tw + toks], neg_inf)
                es = [jnp.exp(mm - ms[0]) for mm in ms]
                tot = es[0]
                for e in es[1:]:
                    tot = tot + e
                inv = 1.0 / tot
                for r in range(TOP_K):
                    rcol = jnp.full((L,), r, dtype=jnp.int32)
                    plsc.store_scatter(obuf_s, [otoks, rcol], es[r] * inv)
                    plsc.store_scatter(obuf_i, [otoks, rcol], ixs[r])
                return carry
            return group_body

        nsb = tw // sub
        flushes = []
        for sb in range(nsb):
            half = sb % 2
            if sb >= 2:
                flushes[2 * (sb - 2)].wait()
                flushes[2 * (sb - 2) + 1].wait()
            lax.fori_loop(0, groups, make_group_body(sb, half), 0)
            dst = pl.ds(t0 + sb * sub, sub)
            flushes.append(pltpu.async_copy(
                obuf_s.at[pl.ds(half * sub, sub), :],
                scores_ref.at[dst, :], sems.at[half]))
            flushes.append(pltpu.async_copy(
                obuf_i.at[pl.ds(half * sub, sub), :],
                idx_ref.at[dst, :], sems.at[half]))
        for cp in flushes[2 * max(0, nsb - 2):]:
            cp.wait()

    return pl.kernel(
        body,
        out_type=[
            jax.ShapeDtypeStruct((tokens, TOP_K), jnp.float32),
            jax.ShapeDtypeStruct((tokens, TOP_K), jnp.int32),
        ],
        mesh=plsc.VectorSubcoreMesh(core_axis_name="c", subcore_axis_name="s"),
        compiler_params=pltpu.CompilerParams(needs_layout_passes=False),
        scratch_types=[
            pltpu.VMEM((experts * tw,), jnp.float32),
            pltpu.VMEM((2 * sub, TOP_K), jnp.float32),
            pltpu.VMEM((2 * sub, TOP_K), jnp.int32),
            pltpu.SemaphoreType.DMA,
            pltpu.SemaphoreType.DMA((2,)),
        ],
    )


def _pack_body(s_ref, i_ref, so_ref, io_ref):
    bt = so_ref.shape[0]
    so_ref[...] = s_ref[...].reshape(bt, TOP_K)
    io_ref[...] = i_ref[...].reshape(bt, TOP_K)


def _pack_outputs(scores_flat, idx_flat, tokens, block_tokens=2048):
    return pl.pallas_call(
        _pack_body,
        grid=(tokens // block_tokens,),
        in_specs=[pl.BlockSpec((block_tokens * TOP_K,), lambda i: (i,)),
                  pl.BlockSpec((block_tokens * TOP_K,), lambda i: (i,))],
        out_specs=[pl.BlockSpec((block_tokens, TOP_K), lambda i: (i, 0)),
                   pl.BlockSpec((block_tokens, TOP_K), lambda i: (i, 0))],
        out_shape=[jax.ShapeDtypeStruct((tokens, TOP_K), jnp.float32),
                   jax.ShapeDtypeStruct((tokens, TOP_K), jnp.int32)],
        compiler_params=pltpu.CompilerParams(
            dimension_semantics=("arbitrary",),
        ),
    )(scores_flat, idx_flat)


NUM_CHUNKS = 1


@jax.jit
def kernel(hidden_states, weight, bias):
    tokens, _ = hidden_states.shape
    experts = weight.shape[0]
    w_t = weight.T
    bias2d = bias.reshape(1, experts)
    logits, logits_t = _router_logits(hidden_states, w_t, bias2d,
                                      None, block_tokens=512,
                                      chunk_tokens=tokens, chunk=0)
    scores, indices = _make_topk_sc(tokens, experts)(logits_t)
    return logits, scores, indices
